# baseline (speedup 1.0000x reference)
import functools

import jax
import jax.numpy as jnp
from jax import lax
from jax.experimental import pallas as pl
from jax.experimental.pallas import tpu as pltpu
from jax.experimental.pallas import tpu_sc as plsc

D = 32
L = 16
NC = 2
NS = 16
NW = NC * NS
SUB = 128
NBUF = 2
PACK = 128 // D
PD = PACK * D
GRP = PACK * SUB


@functools.lru_cache(maxsize=None)
def _build_pack_kernel(n_nodes: int):
    steps = (n_nodes + GRP - 1) // GRP
    n_packed = steps * SUB

    def pack_body(q0, q1, q2, q3, out):
        out[...] = jnp.concatenate(
            [q0[...].T, q1[...].T, q2[...].T, q3[...].T], axis=1)

    in_specs = [
        pl.BlockSpec((D, SUB), functools.partial(
            lambda k, i: (0, PACK * i + k), k))
        for k in range(PACK)
    ]
    return pl.pallas_call(
        pack_body,
        grid=(steps,),
        in_specs=in_specs,
        out_specs=pl.BlockSpec((SUB, PD), lambda i: (i, 0)),
        out_shape=jax.ShapeDtypeStruct((n_packed, PD), jnp.float32),
    ), n_packed


@functools.lru_cache(maxsize=None)
def _build_score_kernel(total: int, n_packed: int, n_rels: int):
    per_w = total // NW
    nsub = per_w // SUB
    mesh = plsc.VectorSubcoreMesh(core_axis_name="c", subcore_axis_name="s")

    @functools.partial(
        pl.kernel,
        mesh=mesh,
        compiler_params=pltpu.CompilerParams(needs_layout_passes=False,
                                             use_tc_tiling_on_sc=True),
        out_type=jax.ShapeDtypeStruct((total,), jnp.float32),
        scratch_types=[
            pltpu.VMEM((per_w,), jnp.int32),
            pltpu.VMEM((per_w,), jnp.int32),
            pltpu.VMEM((per_w,), jnp.int32),
            pltpu.VMEM((per_w,), jnp.int32),
            pltpu.VMEM((per_w,), jnp.int32),
            pltpu.VMEM((n_rels // PACK, PD), jnp.float32),
            pltpu.VMEM((SUB, PD), jnp.float32),
            pltpu.VMEM((SUB, PD), jnp.float32),
            pltpu.VMEM((SUB, PD), jnp.float32),
            pltpu.VMEM((SUB, PD), jnp.float32),
            pltpu.VMEM((per_w,), jnp.float32),
            pltpu.SemaphoreType.DMA,
            pltpu.SemaphoreType.DMA,
        ],
    )
    def score_kernel(head_hbm, tail_hbm, rel_hbm, itab_hbm, rtab_hbm,
                     out_hbm, hidx, tidx, ridx, hidxp, tidxp,
                     rtab_v, hrows0, hrows1, trows0, trows1,
                     acc, sem0, sem1):
        sems = [sem0, sem1]
        hrows = [hrows0, hrows1]
        trows = [trows0, trows1]
        wid = lax.axis_index("s") * NC + lax.axis_index("c")
        base = wid * per_w

        pltpu.sync_copy(head_hbm.at[pl.ds(base, per_w)], hidx)
        pltpu.sync_copy(tail_hbm.at[pl.ds(base, per_w)], tidx)
        pltpu.sync_copy(rel_hbm.at[pl.ds(base, per_w)], ridx)
        pltpu.sync_copy(rtab_hbm, rtab_v)

        low = jnp.full((L,), SUB - 1, jnp.int32)

        def packed_row(v):
            return lax.shift_left(lax.shift_right_logical(v, 9), 7) | \
                (v & low)

        def pack_body(k, carry):
            hv = hidx[pl.ds(k * L, L)]
            tv = tidx[pl.ds(k * L, L)]
            hidxp[pl.ds(k * L, L)] = packed_row(hv)
            tidxp[pl.ds(k * L, L)] = packed_row(tv)
            return carry

        lax.fori_loop(0, per_w // L, pack_body, 0)

        def start(c, b):
            off = c * SUB
            pltpu.make_async_copy(itab_hbm.at[hidxp.at[pl.ds(off, SUB)]],
                                  hrows[b], sems[b]).start()
            pltpu.make_async_copy(itab_hbm.at[tidxp.at[pl.ds(off, SUB)]],
                                  trows[b], sems[b]).start()

        def wait(c, b):
            pltpu.make_async_copy(itab_hbm.at[hidxp.at[pl.ds(c * SUB, SUB)]],
                                  hrows[b], sems[b]).wait()
            pltpu.make_async_copy(itab_hbm.at[tidxp.at[pl.ds(c * SUB, SUB)]],
                                  trows[b], sems[b]).wait()

        for b in range(NBUF):
            start(b, b)

        lane = lax.iota(jnp.int32, L)
        three = jnp.full((L,), PACK - 1, jnp.int32)

        def compute(c, b):
            def group_body(g, carry):
                pos = c * SUB + g * L
                rows = g * L + lane
                hraw = hidx[pl.ds(pos, L)]
                traw = tidx[pl.ds(pos, L)]
                rraw = ridx[pl.ds(pos, L)]
                hoff = lax.shift_left(
                    lax.shift_right_logical(hraw, 7) & three, 5)
                toff = lax.shift_left(
                    lax.shift_right_logical(traw, 7) & three, 5)
                rrow = lax.shift_right_logical(rraw, 2)
                roff = lax.shift_left(rraw & three, 5)
                s = jnp.zeros((L,), jnp.float32)
                for j in range(D):
                    h = plsc.load_gather(hrows[b], [rows, hoff + j])
                    t = plsc.load_gather(trows[b], [rows, toff + j])
                    r = plsc.load_gather(rtab_v, [rrow, roff + j])
                    d = h + r - t
                    s = s + d * d
                acc[pl.ds(pos, L)] = -s
                return carry

            lax.fori_loop(0, SUB // L, group_body, 0)

        def body(i, carry):
            for b in range(NBUF):
                c = i * NBUF + b
                wait(c, b)
                compute(c, b)
                nxt = c + NBUF

                @pl.when(nxt < nsub)
                def _():
                    start(nxt, b)

            return carry

        lax.fori_loop(0, nsub // NBUF, body, 0)
        pltpu.sync_copy(acc, out_hbm.at[pl.ds(base, per_w)])

    return score_kernel


def kernel(head_ids, tail_ids, relation_ids, i_embeddings, r_embeddings):
    B, K = head_ids.shape
    total = B * K
    h1 = head_ids.astype(jnp.int32).reshape(-1)
    t1 = tail_ids.astype(jnp.int32).reshape(-1)
    r1 = relation_ids.astype(jnp.int32).reshape(-1)
    pack, n_packed = _build_pack_kernel(i_embeddings.shape[0])
    tab_t = i_embeddings.T
    itab_packed = pack(tab_t, tab_t, tab_t, tab_t)
    rtab_packed = r_embeddings.reshape(r_embeddings.shape[0] // PACK, PD)
    score = _build_score_kernel(total, n_packed, r_embeddings.shape[0])
    out = score(h1, t1, r1, itab_packed, rtab_packed)
    prediction = out.reshape(B, K)
    pos_pred = prediction[:, :2].reshape(-1)
    neg_pred = prediction[:, 2:].reshape(-1)
    return prediction, pos_pred, neg_pred


# speedup vs baseline: 3.0157x; 3.0157x over previous
import functools

import jax
import jax.numpy as jnp
from jax import lax
from jax.experimental import pallas as pl
from jax.experimental.pallas import tpu as pltpu
from jax.experimental.pallas import tpu_sc as plsc

D = 32
L = 16
NC = 2
NS = 16
NW = NC * NS
SUB = 128
NBUF = 2
PACK = 128 // D
PD = PACK * D
QC = 2048
QSH = 11
GRP = PACK * QC


@functools.lru_cache(maxsize=None)
def _build_pack_kernel(n_nodes: int):
    steps = (n_nodes + GRP - 1) // GRP
    n_packed = steps * QC

    def pack_body(x, out):
        cols = []
        for k in range(PACK):
            cols.append(x[:, pl.ds(k * QC, QC)].T)
        out[...] = jnp.concatenate(cols, axis=1)

    return pl.pallas_call(
        pack_body,
        grid=(steps,),
        in_specs=[pl.BlockSpec((D, GRP), lambda i: (0, i))],
        out_specs=pl.BlockSpec((QC, PD), lambda i: (i, 0)),
        out_shape=jax.ShapeDtypeStruct((n_packed, PD), jnp.float32),
    ), n_packed


@functools.lru_cache(maxsize=None)
def _build_score_kernel(total: int, n_packed: int, n_rels: int):
    per_w = total // NW
    nsub = per_w // SUB
    mesh = plsc.VectorSubcoreMesh(core_axis_name="c", subcore_axis_name="s")

    @functools.partial(
        pl.kernel,
        mesh=mesh,
        compiler_params=pltpu.CompilerParams(needs_layout_passes=False,
                                             use_tc_tiling_on_sc=True),
        out_type=jax.ShapeDtypeStruct((total,), jnp.float32),
        scratch_types=[
            pltpu.VMEM((per_w,), jnp.int32),
            pltpu.VMEM((per_w,), jnp.int32),
            pltpu.VMEM((per_w,), jnp.int32),
            pltpu.VMEM((per_w,), jnp.int32),
            pltpu.VMEM((per_w,), jnp.int32),
            pltpu.VMEM((n_rels // PACK, PD), jnp.float32),
            pltpu.VMEM((SUB, PD), jnp.float32),
            pltpu.VMEM((SUB, PD), jnp.float32),
            pltpu.VMEM((SUB, PD), jnp.float32),
            pltpu.VMEM((SUB, PD), jnp.float32),
            pltpu.VMEM((per_w,), jnp.float32),
            pltpu.SemaphoreType.DMA,
            pltpu.SemaphoreType.DMA,
        ],
    )
    def score_kernel(head_hbm, tail_hbm, rel_hbm, itab_hbm, rtab_hbm,
                     out_hbm, hidx, tidx, ridx, hidxp, tidxp,
                     rtab_v, hrows0, hrows1, trows0, trows1,
                     acc, sem0, sem1):
        sems = [sem0, sem1]
        hrows = [hrows0, hrows1]
        trows = [trows0, trows1]
        wid = lax.axis_index("s") * NC + lax.axis_index("c")
        base = wid * per_w

        pltpu.sync_copy(head_hbm.at[pl.ds(base, per_w)], hidx)
        pltpu.sync_copy(tail_hbm.at[pl.ds(base, per_w)], tidx)
        pltpu.sync_copy(rel_hbm.at[pl.ds(base, per_w)], ridx)
        pltpu.sync_copy(rtab_hbm, rtab_v)

        low = jnp.full((L,), QC - 1, jnp.int32)

        def packed_row(v):
            return lax.shift_left(lax.shift_right_logical(v, QSH + 2),
                                  QSH) | (v & low)

        def pack_body(k, carry):
            hv = hidx[pl.ds(k * L, L)]
            tv = tidx[pl.ds(k * L, L)]
            hidxp[pl.ds(k * L, L)] = packed_row(hv)
            tidxp[pl.ds(k * L, L)] = packed_row(tv)
            return carry

        lax.fori_loop(0, per_w // L, pack_body, 0)

        def start(c, b):
            off = c * SUB
            pltpu.make_async_copy(itab_hbm.at[hidxp.at[pl.ds(off, SUB)]],
                                  hrows[b], sems[b]).start()
            pltpu.make_async_copy(itab_hbm.at[tidxp.at[pl.ds(off, SUB)]],
                                  trows[b], sems[b]).start()

        def wait(c, b):
            pltpu.make_async_copy(itab_hbm.at[hidxp.at[pl.ds(c * SUB, SUB)]],
                                  hrows[b], sems[b]).wait()
            pltpu.make_async_copy(itab_hbm.at[tidxp.at[pl.ds(c * SUB, SUB)]],
                                  trows[b], sems[b]).wait()

        for b in range(NBUF):
            start(b, b)

        lane = lax.iota(jnp.int32, L)
        three = jnp.full((L,), PACK - 1, jnp.int32)

        def compute(c, b):
            def group_body(g, carry):
                pos = c * SUB + g * L
                rows = g * L + lane
                hraw = hidx[pl.ds(pos, L)]
                traw = tidx[pl.ds(pos, L)]
                rraw = ridx[pl.ds(pos, L)]
                hoff = lax.shift_left(
                    lax.shift_right_logical(hraw, QSH) & three, 5)
                toff = lax.shift_left(
                    lax.shift_right_logical(traw, QSH) & three, 5)
                rrow = lax.shift_right_logical(rraw, 2)
                roff = lax.shift_left(rraw & three, 5)
                s = jnp.zeros((L,), jnp.float32)
                for j in range(D):
                    h = plsc.load_gather(hrows[b], [rows, hoff + j])
                    t = plsc.load_gather(trows[b], [rows, toff + j])
                    r = plsc.load_gather(rtab_v, [rrow, roff + j])
                    d = h + r - t
                    s = s + d * d
                acc[pl.ds(pos, L)] = -s
                return carry

            lax.fori_loop(0, SUB // L, group_body, 0)

        def body(i, carry):
            for b in range(NBUF):
                c = i * NBUF + b
                wait(c, b)
                compute(c, b)
                nxt = c + NBUF

                @pl.when(nxt < nsub)
                def _():
                    start(nxt, b)

            return carry

        lax.fori_loop(0, nsub // NBUF, body, 0)
        pltpu.sync_copy(acc, out_hbm.at[pl.ds(base, per_w)])

    return score_kernel


def kernel(head_ids, tail_ids, relation_ids, i_embeddings, r_embeddings):
    B, K = head_ids.shape
    total = B * K
    h1 = head_ids.astype(jnp.int32).reshape(-1)
    t1 = tail_ids.astype(jnp.int32).reshape(-1)
    r1 = relation_ids.astype(jnp.int32).reshape(-1)
    pack, n_packed = _build_pack_kernel(i_embeddings.shape[0])
    itab_packed = pack(i_embeddings.T)
    rtab_packed = r_embeddings.reshape(r_embeddings.shape[0] // PACK, PD)
    score = _build_score_kernel(total, n_packed, r_embeddings.shape[0])
    out = score(h1, t1, r1, itab_packed, rtab_packed)
    prediction = out.reshape(B, K)
    pos_pred = prediction[:, :2].reshape(-1)
    neg_pred = prediction[:, 2:].reshape(-1)
    return prediction, pos_pred, neg_pred


# speedup vs baseline: 3.0393x; 1.0078x over previous
import functools

import jax
import jax.numpy as jnp
from jax import lax
from jax.experimental import pallas as pl
from jax.experimental.pallas import tpu as pltpu
from jax.experimental.pallas import tpu_sc as plsc

D = 32
L = 16
NC = 2
NS = 16
NW = NC * NS
SUB = 128
NBUF = 2
PACK = 128 // D
PD = PACK * D
QC = 4096
QSH = 12
GRP = PACK * QC


@functools.lru_cache(maxsize=None)
def _build_pack_kernel(n_nodes: int):
    steps = (n_nodes + GRP - 1) // GRP
    n_packed = steps * QC

    def pack_body(x, out):
        cols = []
        for k in range(PACK):
            cols.append(x[:, pl.ds(k * QC, QC)].T)
        out[...] = jnp.concatenate(cols, axis=1)

    return pl.pallas_call(
        pack_body,
        grid=(steps,),
        in_specs=[pl.BlockSpec((D, GRP), lambda i: (0, i))],
        out_specs=pl.BlockSpec((QC, PD), lambda i: (i, 0)),
        out_shape=jax.ShapeDtypeStruct((n_packed, PD), jnp.float32),
    ), n_packed


@functools.lru_cache(maxsize=None)
def _build_score_kernel(total: int, n_packed: int, n_rels: int):
    per_w = total // NW
    nsub = per_w // SUB
    mesh = plsc.VectorSubcoreMesh(core_axis_name="c", subcore_axis_name="s")

    @functools.partial(
        pl.kernel,
        mesh=mesh,
        compiler_params=pltpu.CompilerParams(needs_layout_passes=False,
                                             use_tc_tiling_on_sc=True),
        out_type=jax.ShapeDtypeStruct((total,), jnp.float32),
        scratch_types=[
            pltpu.VMEM((per_w,), jnp.int32),
            pltpu.VMEM((per_w,), jnp.int32),
            pltpu.VMEM((per_w,), jnp.int32),
            pltpu.VMEM((per_w,), jnp.int32),
            pltpu.VMEM((per_w,), jnp.int32),
            pltpu.VMEM((n_rels // PACK, PD), jnp.float32),
            pltpu.VMEM((SUB, PD), jnp.float32),
            pltpu.VMEM((SUB, PD), jnp.float32),
            pltpu.VMEM((SUB, PD), jnp.float32),
            pltpu.VMEM((SUB, PD), jnp.float32),
            pltpu.VMEM((per_w,), jnp.float32),
            pltpu.SemaphoreType.DMA,
            pltpu.SemaphoreType.DMA,
        ],
    )
    def score_kernel(head_hbm, tail_hbm, rel_hbm, itab_hbm, rtab_hbm,
                     out_hbm, hidx, tidx, ridx, hidxp, tidxp,
                     rtab_v, hrows0, hrows1, trows0, trows1,
                     acc, sem0, sem1):
        sems = [sem0, sem1]
        hrows = [hrows0, hrows1]
        trows = [trows0, trows1]
        wid = lax.axis_index("s") * NC + lax.axis_index("c")
        base = wid * per_w

        pltpu.sync_copy(head_hbm.at[pl.ds(base, per_w)], hidx)
        pltpu.sync_copy(tail_hbm.at[pl.ds(base, per_w)], tidx)
        pltpu.sync_copy(rel_hbm.at[pl.ds(base, per_w)], ridx)
        pltpu.sync_copy(rtab_hbm, rtab_v)

        low = jnp.full((L,), QC - 1, jnp.int32)

        def packed_row(v):
            return lax.shift_left(lax.shift_right_logical(v, QSH + 2),
                                  QSH) | (v & low)

        def pack_body(k, carry):
            hv = hidx[pl.ds(k * L, L)]
            tv = tidx[pl.ds(k * L, L)]
            hidxp[pl.ds(k * L, L)] = packed_row(hv)
            tidxp[pl.ds(k * L, L)] = packed_row(tv)
            return carry

        lax.fori_loop(0, per_w // L, pack_body, 0)

        def start(c, b):
            off = c * SUB
            pltpu.make_async_copy(itab_hbm.at[hidxp.at[pl.ds(off, SUB)]],
                                  hrows[b], sems[b]).start()
            pltpu.make_async_copy(itab_hbm.at[tidxp.at[pl.ds(off, SUB)]],
                                  trows[b], sems[b]).start()

        def wait(c, b):
            pltpu.make_async_copy(itab_hbm.at[hidxp.at[pl.ds(c * SUB, SUB)]],
                                  hrows[b], sems[b]).wait()
            pltpu.make_async_copy(itab_hbm.at[tidxp.at[pl.ds(c * SUB, SUB)]],
                                  trows[b], sems[b]).wait()

        for b in range(NBUF):
            start(b, b)

        lane = lax.iota(jnp.int32, L)
        three = jnp.full((L,), PACK - 1, jnp.int32)

        def compute(c, b):
            def group_body(g, carry):
                pos = c * SUB + g * L
                rows = g * L + lane
                hraw = hidx[pl.ds(pos, L)]
                traw = tidx[pl.ds(pos, L)]
                rraw = ridx[pl.ds(pos, L)]
                hoff = lax.shift_left(
                    lax.shift_right_logical(hraw, QSH) & three, 5)
                toff = lax.shift_left(
                    lax.shift_right_logical(traw, QSH) & three, 5)
                rrow = lax.shift_right_logical(rraw, 2)
                roff = lax.shift_left(rraw & three, 5)
                s = jnp.zeros((L,), jnp.float32)
                for j in range(D):
                    h = plsc.load_gather(hrows[b], [rows, hoff + j])
                    t = plsc.load_gather(trows[b], [rows, toff + j])
                    r = plsc.load_gather(rtab_v, [rrow, roff + j])
                    d = h + r - t
                    s = s + d * d
                acc[pl.ds(pos, L)] = -s
                return carry

            lax.fori_loop(0, SUB // L, group_body, 0)

        def body(i, carry):
            for b in range(NBUF):
                c = i * NBUF + b
                wait(c, b)
                compute(c, b)
                nxt = c + NBUF

                @pl.when(nxt < nsub)
                def _():
                    start(nxt, b)

            return carry

        lax.fori_loop(0, nsub // NBUF, body, 0)
        pltpu.sync_copy(acc, out_hbm.at[pl.ds(base, per_w)])

    return score_kernel


def kernel(head_ids, tail_ids, relation_ids, i_embeddings, r_embeddings):
    B, K = head_ids.shape
    total = B * K
    h1 = head_ids.astype(jnp.int32).reshape(-1)
    t1 = tail_ids.astype(jnp.int32).reshape(-1)
    r1 = relation_ids.astype(jnp.int32).reshape(-1)
    pack, n_packed = _build_pack_kernel(i_embeddings.shape[0])
    itab_packed = pack(i_embeddings.T)
    rtab_packed = r_embeddings.reshape(r_embeddings.shape[0] // PACK, PD)
    score = _build_score_kernel(total, n_packed, r_embeddings.shape[0])
    out = score(h1, t1, r1, itab_packed, rtab_packed)
    prediction = out.reshape(B, K)
    pos_pred = prediction[:, :2].reshape(-1)
    neg_pred = prediction[:, 2:].reshape(-1)
    return prediction, pos_pred, neg_pred


# speedup vs baseline: 3.0397x; 1.0001x over previous
import functools

import jax
import jax.numpy as jnp
from jax import lax
from jax.experimental import pallas as pl
from jax.experimental.pallas import tpu as pltpu
from jax.experimental.pallas import tpu_sc as plsc

D = 32
L = 16
NC = 2
NS = 16
NW = NC * NS
SUB = 64
NBUF = 4
PACK = 128 // D
PD = PACK * D
QC = 4096
QSH = 12
GRP = PACK * QC


@functools.lru_cache(maxsize=None)
def _build_pack_kernel(n_nodes: int):
    steps = (n_nodes + GRP - 1) // GRP
    n_packed = steps * QC

    def pack_body(x, out):
        cols = []
        for k in range(PACK):
            cols.append(x[:, pl.ds(k * QC, QC)].T)
        out[...] = jnp.concatenate(cols, axis=1)

    return pl.pallas_call(
        pack_body,
        grid=(steps,),
        in_specs=[pl.BlockSpec((D, GRP), lambda i: (0, i))],
        out_specs=pl.BlockSpec((QC, PD), lambda i: (i, 0)),
        out_shape=jax.ShapeDtypeStruct((n_packed, PD), jnp.float32),
    ), n_packed


@functools.lru_cache(maxsize=None)
def _build_score_kernel(total: int, n_packed: int, n_rels: int):
    per_w = total // NW
    nsub = per_w // SUB
    mesh = plsc.VectorSubcoreMesh(core_axis_name="c", subcore_axis_name="s")

    @functools.partial(
        pl.kernel,
        mesh=mesh,
        compiler_params=pltpu.CompilerParams(needs_layout_passes=False,
                                             use_tc_tiling_on_sc=True),
        out_type=jax.ShapeDtypeStruct((total,), jnp.float32),
        scratch_types=[
            pltpu.VMEM((per_w,), jnp.int32),
            pltpu.VMEM((per_w,), jnp.int32),
            pltpu.VMEM((per_w,), jnp.int32),
            pltpu.VMEM((per_w,), jnp.int32),
            pltpu.VMEM((per_w,), jnp.int32),
            pltpu.VMEM((n_rels // PACK, PD), jnp.float32),
            pltpu.VMEM((SUB, PD), jnp.float32),
            pltpu.VMEM((SUB, PD), jnp.float32),
            pltpu.VMEM((SUB, PD), jnp.float32),
            pltpu.VMEM((SUB, PD), jnp.float32),
            pltpu.VMEM((SUB, PD), jnp.float32),
            pltpu.VMEM((SUB, PD), jnp.float32),
            pltpu.VMEM((SUB, PD), jnp.float32),
            pltpu.VMEM((SUB, PD), jnp.float32),
            pltpu.VMEM((per_w,), jnp.float32),
            pltpu.SemaphoreType.DMA,
            pltpu.SemaphoreType.DMA,
            pltpu.SemaphoreType.DMA,
            pltpu.SemaphoreType.DMA,
        ],
    )
    def score_kernel(head_hbm, tail_hbm, rel_hbm, itab_hbm, rtab_hbm,
                     out_hbm, hidx, tidx, ridx, hidxp, tidxp,
                     rtab_v, hrows0, hrows1, hrows2, hrows3,
                     trows0, trows1, trows2, trows3,
                     acc, sem0, sem1, sem2, sem3):
        sems = [sem0, sem1, sem2, sem3]
        hrows = [hrows0, hrows1, hrows2, hrows3]
        trows = [trows0, trows1, trows2, trows3]
        wid = lax.axis_index("s") * NC + lax.axis_index("c")
        base = wid * per_w

        pltpu.sync_copy(head_hbm.at[pl.ds(base, per_w)], hidx)
        pltpu.sync_copy(tail_hbm.at[pl.ds(base, per_w)], tidx)
        pltpu.sync_copy(rel_hbm.at[pl.ds(base, per_w)], ridx)
        pltpu.sync_copy(rtab_hbm, rtab_v)

        low = jnp.full((L,), QC - 1, jnp.int32)

        def packed_row(v):
            return lax.shift_left(lax.shift_right_logical(v, QSH + 2),
                                  QSH) | (v & low)

        def pack_body(k, carry):
            hv = hidx[pl.ds(k * L, L)]
            tv = tidx[pl.ds(k * L, L)]
            hidxp[pl.ds(k * L, L)] = packed_row(hv)
            tidxp[pl.ds(k * L, L)] = packed_row(tv)
            return carry

        lax.fori_loop(0, per_w // L, pack_body, 0)

        def start(c, b):
            off = c * SUB
            pltpu.make_async_copy(itab_hbm.at[hidxp.at[pl.ds(off, SUB)]],
                                  hrows[b], sems[b]).start()
            pltpu.make_async_copy(itab_hbm.at[tidxp.at[pl.ds(off, SUB)]],
                                  trows[b], sems[b]).start()

        def wait(c, b):
            pltpu.make_async_copy(itab_hbm.at[hidxp.at[pl.ds(c * SUB, SUB)]],
                                  hrows[b], sems[b]).wait()
            pltpu.make_async_copy(itab_hbm.at[tidxp.at[pl.ds(c * SUB, SUB)]],
                                  trows[b], sems[b]).wait()

        for b in range(NBUF):
            start(b, b)

        lane = lax.iota(jnp.int32, L)
        three = jnp.full((L,), PACK - 1, jnp.int32)

        def compute(c, b):
            def group_body(g, carry):
                pos = c * SUB + g * L
                rows = g * L + lane
                hraw = hidx[pl.ds(pos, L)]
                traw = tidx[pl.ds(pos, L)]
                rraw = ridx[pl.ds(pos, L)]
                hoff = lax.shift_left(
                    lax.shift_right_logical(hraw, QSH) & three, 5)
                toff = lax.shift_left(
                    lax.shift_right_logical(traw, QSH) & three, 5)
                rrow = lax.shift_right_logical(rraw, 2)
                roff = lax.shift_left(rraw & three, 5)
                s = jnp.zeros((L,), jnp.float32)
                for j in range(D):
                    h = plsc.load_gather(hrows[b], [rows, hoff + j])
                    t = plsc.load_gather(trows[b], [rows, toff + j])
                    r = plsc.load_gather(rtab_v, [rrow, roff + j])
                    d = h + r - t
                    s = s + d * d
                acc[pl.ds(pos, L)] = -s
                return carry

            lax.fori_loop(0, SUB // L, group_body, 0)

        def body(i, carry):
            for b in range(NBUF):
                c = i * NBUF + b
                wait(c, b)
                compute(c, b)
                nxt = c + NBUF

                @pl.when(nxt < nsub)
                def _():
                    start(nxt, b)

            return carry

        lax.fori_loop(0, nsub // NBUF, body, 0)
        pltpu.sync_copy(acc, out_hbm.at[pl.ds(base, per_w)])

    return score_kernel


def kernel(head_ids, tail_ids, relation_ids, i_embeddings, r_embeddings):
    B, K = head_ids.shape
    total = B * K
    h1 = head_ids.astype(jnp.int32).reshape(-1)
    t1 = tail_ids.astype(jnp.int32).reshape(-1)
    r1 = relation_ids.astype(jnp.int32).reshape(-1)
    pack, n_packed = _build_pack_kernel(i_embeddings.shape[0])
    itab_packed = pack(i_embeddings.T)
    rtab_packed = r_embeddings.reshape(r_embeddings.shape[0] // PACK, PD)
    score = _build_score_kernel(total, n_packed, r_embeddings.shape[0])
    out = score(h1, t1, r1, itab_packed, rtab_packed)
    prediction = out.reshape(B, K)
    pos_pred = prediction[:, :2].reshape(-1)
    neg_pred = prediction[:, 2:].reshape(-1)
    return prediction, pos_pred, neg_pred


# speedup vs baseline: 3.0419x; 1.0007x over previous
import functools

import jax
import jax.numpy as jnp
from jax import lax
from jax.experimental import pallas as pl
from jax.experimental.pallas import tpu as pltpu
from jax.experimental.pallas import tpu_sc as plsc

D = 32
L = 16
NC = 2
NS = 16
NW = NC * NS
SUB = 128
NBUF = 2
PACK = 128 // D
PD = PACK * D
QC = 4096
QSH = 12
GRP = PACK * QC


@functools.lru_cache(maxsize=None)
def _build_pack_kernel(n_nodes: int):
    steps = (n_nodes + GRP - 1) // GRP
    n_packed = steps * QC

    def pack_body(x, out):
        cols = []
        for k in range(PACK):
            cols.append(x[:, pl.ds(k * QC, QC)].T)
        out[...] = jnp.concatenate(cols, axis=1)

    return pl.pallas_call(
        pack_body,
        grid=(steps,),
        in_specs=[pl.BlockSpec((D, GRP), lambda i: (0, i))],
        out_specs=pl.BlockSpec((QC, PD), lambda i: (i, 0)),
        out_shape=jax.ShapeDtypeStruct((n_packed, PD), jnp.float32),
    ), n_packed


@functools.lru_cache(maxsize=None)
def _build_score_kernel(total: int, n_packed: int, n_rels: int):
    per_w = total // NW
    nsub = per_w // SUB
    mesh = plsc.VectorSubcoreMesh(core_axis_name="c", subcore_axis_name="s")

    @functools.partial(
        pl.kernel,
        mesh=mesh,
        compiler_params=pltpu.CompilerParams(needs_layout_passes=False,
                                             use_tc_tiling_on_sc=True),
        out_type=jax.ShapeDtypeStruct((total,), jnp.float32),
        scratch_types=[
            pltpu.VMEM((per_w,), jnp.int32),
            pltpu.VMEM((per_w,), jnp.int32),
            pltpu.VMEM((per_w,), jnp.int32),
            pltpu.VMEM((per_w,), jnp.int32),
            pltpu.VMEM((per_w,), jnp.int32),
            pltpu.VMEM((n_rels // PACK, PD), jnp.float32),
            pltpu.VMEM((SUB, PD), jnp.float32),
            pltpu.VMEM((SUB, PD), jnp.float32),
            pltpu.VMEM((SUB, PD), jnp.float32),
            pltpu.VMEM((SUB, PD), jnp.float32),
            pltpu.VMEM((per_w,), jnp.float32),
            pltpu.SemaphoreType.DMA,
            pltpu.SemaphoreType.DMA,
        ],
    )
    def score_kernel(head_hbm, tail_hbm, rel_hbm, itab_hbm, rtab_hbm,
                     out_hbm, hidx, tidx, ridx, hidxp, tidxp,
                     rtab_v, hrows0, hrows1, trows0, trows1,
                     acc, sem0, sem1):
        sems = [sem0, sem1]
        hrows = [hrows0, hrows1]
        trows = [trows0, trows1]
        wid = lax.axis_index("s") * NC + lax.axis_index("c")
        base = wid * per_w

        pltpu.sync_copy(head_hbm.at[pl.ds(base, per_w)], hidx)
        pltpu.sync_copy(tail_hbm.at[pl.ds(base, per_w)], tidx)
        pltpu.sync_copy(rel_hbm.at[pl.ds(base, per_w)], ridx)
        pltpu.sync_copy(rtab_hbm, rtab_v)

        low = jnp.full((L,), QC - 1, jnp.int32)

        def packed_row(v):
            return lax.shift_left(lax.shift_right_logical(v, QSH + 2),
                                  QSH) | (v & low)

        def pack_body(k, carry):
            hv = hidx[pl.ds(k * L, L)]
            tv = tidx[pl.ds(k * L, L)]
            hidxp[pl.ds(k * L, L)] = packed_row(hv)
            tidxp[pl.ds(k * L, L)] = packed_row(tv)
            return carry

        lax.fori_loop(0, per_w // L, pack_body, 0)

        def start(c, b):
            off = c * SUB
            pltpu.make_async_copy(itab_hbm.at[hidxp.at[pl.ds(off, SUB)]],
                                  hrows[b], sems[b]).start()
            pltpu.make_async_copy(itab_hbm.at[tidxp.at[pl.ds(off, SUB)]],
                                  trows[b], sems[b]).start()

        def wait(c, b):
            pltpu.make_async_copy(itab_hbm.at[hidxp.at[pl.ds(c * SUB, SUB)]],
                                  hrows[b], sems[b]).wait()
            pltpu.make_async_copy(itab_hbm.at[tidxp.at[pl.ds(c * SUB, SUB)]],
                                  trows[b], sems[b]).wait()

        for b in range(NBUF):
            start(b, b)

        lane = lax.iota(jnp.int32, L)
        three = jnp.full((L,), PACK - 1, jnp.int32)

        def compute(c, b):
            def group_body(g, carry):
                pos = c * SUB + g * L
                rows = g * L + lane
                hraw = hidx[pl.ds(pos, L)]
                traw = tidx[pl.ds(pos, L)]
                rraw = ridx[pl.ds(pos, L)]
                hoff = lax.shift_left(
                    lax.shift_right_logical(hraw, QSH) & three, 5)
                toff = lax.shift_left(
                    lax.shift_right_logical(traw, QSH) & three, 5)
                rrow = lax.shift_right_logical(rraw, 2)
                roff = lax.shift_left(rraw & three, 5)
                s = jnp.zeros((L,), jnp.float32)
                for j in range(D):
                    h = plsc.load_gather(hrows[b], [rows, hoff + j])
                    t = plsc.load_gather(trows[b], [rows, toff + j])
                    r = plsc.load_gather(rtab_v, [rrow, roff + j])
                    d = h + r - t
                    s = s + d * d
                acc[pl.ds(pos, L)] = -s
                return carry

            lax.fori_loop(0, SUB // L, group_body, 0)

        def body(i, carry):
            for b in range(NBUF):
                c = i * NBUF + b
                wait(c, b)
                compute(c, b)
                nxt = c + NBUF

                @pl.when(nxt < nsub)
                def _():
                    start(nxt, b)

            return carry

        lax.fori_loop(0, nsub // NBUF, body, 0)
        pltpu.sync_copy(acc, out_hbm.at[pl.ds(base, per_w)])

    return score_kernel


def kernel(head_ids, tail_ids, relation_ids, i_embeddings, r_embeddings):
    B, K = head_ids.shape
    total = B * K
    h1 = head_ids.astype(jnp.int32).reshape(-1)
    t1 = tail_ids.astype(jnp.int32).reshape(-1)
    r1 = relation_ids.astype(jnp.int32).reshape(-1)
    pack, n_packed = _build_pack_kernel(i_embeddings.shape[0])
    itab_packed = pack(i_embeddings.T)
    rtab_packed = r_embeddings.reshape(r_embeddings.shape[0] // PACK, PD)
    score = _build_score_kernel(total, n_packed, r_embeddings.shape[0])
    out = score(h1, t1, r1, itab_packed, rtab_packed)
    prediction = out.reshape(B, K)
    pos_pred = prediction[:, :2].reshape(-1)
    neg_pred = prediction[:, 2:].reshape(-1)
    return prediction, pos_pred, neg_pred


# speedup vs baseline: 3.0580x; 1.0053x over previous
import functools

import jax
import jax.numpy as jnp
from jax import lax
from jax.experimental import pallas as pl
from jax.experimental.pallas import tpu as pltpu
from jax.experimental.pallas import tpu_sc as plsc

D = 32
L = 16
NC = 2
NS = 16
NW = NC * NS
SUB = 128
NBUF = 2
PACK = 128 // D
PD = PACK * D
QC = 8192
QSH = 13
GRP = PACK * QC


@functools.lru_cache(maxsize=None)
def _build_pack_kernel(n_nodes: int):
    steps = (n_nodes + GRP - 1) // GRP
    n_packed = steps * QC

    def pack_body(x, out):
        cols = []
        for k in range(PACK):
            cols.append(x[:, pl.ds(k * QC, QC)].T)
        out[...] = jnp.concatenate(cols, axis=1)

    return pl.pallas_call(
        pack_body,
        grid=(steps,),
        in_specs=[pl.BlockSpec((D, GRP), lambda i: (0, i))],
        out_specs=pl.BlockSpec((QC, PD), lambda i: (i, 0)),
        out_shape=jax.ShapeDtypeStruct((n_packed, PD), jnp.float32),
    ), n_packed


@functools.lru_cache(maxsize=None)
def _build_score_kernel(total: int, n_packed: int, n_rels: int):
    per_w = total // NW
    nsub = per_w // SUB
    mesh = plsc.VectorSubcoreMesh(core_axis_name="c", subcore_axis_name="s")

    @functools.partial(
        pl.kernel,
        mesh=mesh,
        compiler_params=pltpu.CompilerParams(needs_layout_passes=False,
                                             use_tc_tiling_on_sc=True),
        out_type=jax.ShapeDtypeStruct((total,), jnp.float32),
        scratch_types=[
            pltpu.VMEM((per_w,), jnp.int32),
            pltpu.VMEM((per_w,), jnp.int32),
            pltpu.VMEM((per_w,), jnp.int32),
            pltpu.VMEM((per_w,), jnp.int32),
            pltpu.VMEM((per_w,), jnp.int32),
            pltpu.VMEM((n_rels // PACK, PD), jnp.float32),
            pltpu.VMEM((SUB, PD), jnp.float32),
            pltpu.VMEM((SUB, PD), jnp.float32),
            pltpu.VMEM((SUB, PD), jnp.float32),
            pltpu.VMEM((SUB, PD), jnp.float32),
            pltpu.VMEM((per_w,), jnp.float32),
            pltpu.SemaphoreType.DMA,
            pltpu.SemaphoreType.DMA,
        ],
    )
    def score_kernel(head_hbm, tail_hbm, rel_hbm, itab_hbm, rtab_hbm,
                     out_hbm, hidx, tidx, ridx, hidxp, tidxp,
                     rtab_v, hrows0, hrows1, trows0, trows1,
                     acc, sem0, sem1):
        sems = [sem0, sem1]
        hrows = [hrows0, hrows1]
        trows = [trows0, trows1]
        wid = lax.axis_index("s") * NC + lax.axis_index("c")
        base = wid * per_w

        pltpu.sync_copy(head_hbm.at[pl.ds(base, per_w)], hidx)
        pltpu.sync_copy(tail_hbm.at[pl.ds(base, per_w)], tidx)
        pltpu.sync_copy(rel_hbm.at[pl.ds(base, per_w)], ridx)
        pltpu.sync_copy(rtab_hbm, rtab_v)

        low = jnp.full((L,), QC - 1, jnp.int32)

        def packed_row(v):
            return lax.shift_left(lax.shift_right_logical(v, QSH + 2),
                                  QSH) | (v & low)

        def pack_body(k, carry):
            hv = hidx[pl.ds(k * L, L)]
            tv = tidx[pl.ds(k * L, L)]
            hidxp[pl.ds(k * L, L)] = packed_row(hv)
            tidxp[pl.ds(k * L, L)] = packed_row(tv)
            return carry

        lax.fori_loop(0, per_w // L, pack_body, 0)

        def start(c, b):
            off = c * SUB
            pltpu.make_async_copy(itab_hbm.at[hidxp.at[pl.ds(off, SUB)]],
                                  hrows[b], sems[b]).start()
            pltpu.make_async_copy(itab_hbm.at[tidxp.at[pl.ds(off, SUB)]],
                                  trows[b], sems[b]).start()

        def wait(c, b):
            pltpu.make_async_copy(itab_hbm.at[hidxp.at[pl.ds(c * SUB, SUB)]],
                                  hrows[b], sems[b]).wait()
            pltpu.make_async_copy(itab_hbm.at[tidxp.at[pl.ds(c * SUB, SUB)]],
                                  trows[b], sems[b]).wait()

        for b in range(NBUF):
            start(b, b)

        lane = lax.iota(jnp.int32, L)
        three = jnp.full((L,), PACK - 1, jnp.int32)

        def compute(c, b):
            def group_body(g, carry):
                pos = c * SUB + g * L
                rows = g * L + lane
                hraw = hidx[pl.ds(pos, L)]
                traw = tidx[pl.ds(pos, L)]
                rraw = ridx[pl.ds(pos, L)]
                hoff = lax.shift_left(
                    lax.shift_right_logical(hraw, QSH) & three, 5)
                toff = lax.shift_left(
                    lax.shift_right_logical(traw, QSH) & three, 5)
                rrow = lax.shift_right_logical(rraw, 2)
                roff = lax.shift_left(rraw & three, 5)
                s = jnp.zeros((L,), jnp.float32)
                for j in range(D):
                    h = plsc.load_gather(hrows[b], [rows, hoff + j])
                    t = plsc.load_gather(trows[b], [rows, toff + j])
                    r = plsc.load_gather(rtab_v, [rrow, roff + j])
                    d = h + r - t
                    s = s + d * d
                acc[pl.ds(pos, L)] = -s
                return carry

            lax.fori_loop(0, SUB // L, group_body, 0)

        def body(i, carry):
            for b in range(NBUF):
                c = i * NBUF + b
                wait(c, b)
                compute(c, b)
                nxt = c + NBUF

                @pl.when(nxt < nsub)
                def _():
                    start(nxt, b)

            return carry

        lax.fori_loop(0, nsub // NBUF, body, 0)
        pltpu.sync_copy(acc, out_hbm.at[pl.ds(base, per_w)])

    return score_kernel


def kernel(head_ids, tail_ids, relation_ids, i_embeddings, r_embeddings):
    B, K = head_ids.shape
    total = B * K
    h1 = head_ids.astype(jnp.int32).reshape(-1)
    t1 = tail_ids.astype(jnp.int32).reshape(-1)
    r1 = relation_ids.astype(jnp.int32).reshape(-1)
    pack, n_packed = _build_pack_kernel(i_embeddings.shape[0])
    itab_packed = pack(i_embeddings.T)
    rtab_packed = r_embeddings.reshape(r_embeddings.shape[0] // PACK, PD)
    score = _build_score_kernel(total, n_packed, r_embeddings.shape[0])
    out = score(h1, t1, r1, itab_packed, rtab_packed)
    prediction = out.reshape(B, K)
    pos_pred = prediction[:, :2].reshape(-1)
    neg_pred = prediction[:, 2:].reshape(-1)
    return prediction, pos_pred, neg_pred
